# Initial kernel scaffold; baseline (speedup 1.0000x reference)
import jax
import jax.numpy as jnp
from jax.experimental import pallas as pl


def kernel(x, Wg, W1, W2):
    raise NotImplementedError("write your pallas kernel here")


# speedup vs baseline: 1.3922x; 1.3922x over previous
import functools

import jax
import jax.numpy as jnp
from jax.experimental import pallas as pl
from jax.experimental.pallas import tpu as pltpu

D_MODEL = 768
FF = 3072
N_EXPERT = 8
TOP_K = 2

F_SPLIT = 2
F_TILE = FF // F_SPLIT


def _moe_dense_kernel(x_ref, wg_ref, w1_ref, w2_ref, out_ref, w_scr):
    e = pl.program_id(0)
    f = pl.program_id(1)

    @pl.when((e == 0) & (f == 0))
    def _init():
        x = x_ref[...]
        scores = jax.lax.dot_general(
            x, wg_ref[...], (((1,), (1,)), ((), ())),
            preferred_element_type=jnp.float32)
        T, E = scores.shape
        iota = jax.lax.broadcasted_iota(jnp.int32, (T, E), 1)
        m1 = jnp.max(scores, axis=-1, keepdims=True)
        idx1 = jnp.min(jnp.where(scores == m1, iota, E), axis=-1, keepdims=True)
        s2 = jnp.where(iota == idx1, -jnp.inf, scores)
        m2 = jnp.max(s2, axis=-1, keepdims=True)
        idx2 = jnp.min(jnp.where(s2 == m2, iota, E), axis=-1, keepdims=True)
        z = jnp.exp(m2 - m1)
        p1 = 1.0 / (1.0 + z)
        p2 = 1.0 - p1
        w_scr[...] = jnp.where(iota == idx1, p1, 0.0) + jnp.where(iota == idx2, p2, 0.0)
        out_ref[...] = jnp.zeros_like(out_ref)

    x = x_ref[...]
    h = jax.lax.dot_general(
        x, w1_ref[0], (((1,), (1,)), ((), ())),
        preferred_element_type=jnp.float32)
    h = h * (1.0 / (1.0 + jnp.exp(-h)))
    o = jax.lax.dot_general(
        h, w2_ref[0], (((1,), (1,)), ((), ())),
        preferred_element_type=jnp.float32)
    w_all = w_scr[...]
    sel = jax.lax.broadcasted_iota(jnp.int32, w_all.shape, 1) == e
    w_col = jnp.sum(jnp.where(sel, w_all, 0.0), axis=-1, keepdims=True)
    out_ref[...] += w_col * o


def kernel(x, Wg, W1, W2):
    B, T, C = x.shape
    x_flat = x.reshape(T, C)
    out = pl.pallas_call(
        _moe_dense_kernel,
        grid=(N_EXPERT, F_SPLIT),
        in_specs=[
            pl.BlockSpec((T, C), lambda e, f: (0, 0)),
            pl.BlockSpec((N_EXPERT, C), lambda e, f: (0, 0)),
            pl.BlockSpec((1, F_TILE, C), lambda e, f: (e, f, 0)),
            pl.BlockSpec((1, C, F_TILE), lambda e, f: (e, 0, f)),
        ],
        out_specs=pl.BlockSpec((T, C), lambda e, f: (0, 0)),
        out_shape=jax.ShapeDtypeStruct((T, C), jnp.float32),
        scratch_shapes=[pltpu.VMEM((T, N_EXPERT), jnp.float32)],
        compiler_params=pltpu.CompilerParams(
            dimension_semantics=("arbitrary", "arbitrary")),
    )(x_flat, Wg, W1, W2)
    return out.reshape(B, T, C)
